# baseline (speedup 1.0000x reference)
import functools

import jax
import jax.numpy as jnp
from jax import lax
from jax.experimental import pallas as pl
from jax.experimental.pallas import tpu as pltpu
from jax.experimental.pallas import tpu_sc as plsc

_B = 65536
_F = 768

_NUM_CORES = 2
_NUM_SUBCORES = 16
_NW = _NUM_CORES * _NUM_SUBCORES
_RCHUNK = 32


def _sc_body(nchunks, l_hbm, r_hbm, o_hbm,
             l0, l1, r0, r1,
             in_l0, in_l1, in_r0, in_r1, out0, out1):
    span = nchunks * _RCHUNK
    wid = lax.axis_index("s") * _NUM_CORES + lax.axis_index("c")
    base = wid * span

    lbuf = (l0, l1)
    rbuf = (r0, r1)
    in_l = (in_l0, in_l1)
    in_r = (in_r0, in_r1)
    out = (out0, out1)

    def l_slice(c):
        return l_hbm.at[pl.ds(base + c * _RCHUNK, _RCHUNK), :]

    def r_slice(c):
        return r_hbm.at[pl.ds(base + c * _RCHUNK, _RCHUNK), :]

    def o_slice(c):
        return o_hbm.at[pl.ds(base + c * _RCHUNK, _RCHUNK), :]

    pltpu.async_copy(l_slice(0), lbuf[0], in_l[0])
    for b in range(2):
        pltpu.async_copy(r_slice(b), rbuf[b], in_r[b])

    @pl.loop(0, nchunks, step=2)
    def _ring(g):
        for b in range(2):
            c = g + b

            @pl.when(c >= 1)
            def _():
                pltpu.make_async_copy(
                    lbuf[1 - b], o_slice(c - 1), out[1 - b]).wait()

            @pl.when(c + 1 < nchunks)
            def _():
                pltpu.async_copy(l_slice(c + 1), lbuf[1 - b], in_l[1 - b])

            pltpu.make_async_copy(l_slice(c), lbuf[b], in_l[b]).wait()
            pltpu.make_async_copy(r_slice(c), rbuf[b], in_r[b]).wait()

            for r in range(_RCHUNK):
                @plsc.parallel_loop(0, _F, step=16, unroll=8)
                def _mul(j):
                    lbuf[b][r, pl.ds(j, 16)] = (
                        lbuf[b][r, pl.ds(j, 16)] * rbuf[b][r, pl.ds(j, 16)]
                    )

            pltpu.async_copy(lbuf[b], o_slice(c), out[b])

            @pl.when(c + 2 < nchunks)
            def _():
                pltpu.async_copy(r_slice(c + 2), rbuf[b], in_r[b])

    pltpu.make_async_copy(lbuf[1], o_slice(nchunks - 1), out[1]).wait()


@functools.cache
def _make_sc_mul(n_rows):
    nchunks = n_rows // (_NW * _RCHUNK)
    assert nchunks * _NW * _RCHUNK == n_rows and nchunks % 2 == 0

    @functools.partial(
        pl.kernel,
        out_type=jax.ShapeDtypeStruct((n_rows, _F), jnp.float32),
        mesh=plsc.VectorSubcoreMesh(core_axis_name="c", subcore_axis_name="s"),
        scratch_types=(
            [pltpu.VMEM((_RCHUNK, _F), jnp.float32)] * 4
            + [pltpu.SemaphoreType.DMA] * 6
        ),
    )
    def sc_mul(l_hbm, r_hbm, o_hbm, *scratch):
        _sc_body(nchunks, l_hbm, r_hbm, o_hbm, *scratch)

    return sc_mul


def kernel(left_input, right_input):
    return _make_sc_mul(_B)(left_input, right_input)


# speedup vs baseline: 1.2935x; 1.2935x over previous
import jax
import jax.numpy as jnp
from jax.experimental import pallas as pl


def _mul_body(l_ref, r_ref, o_ref):
    o_ref[...] = l_ref[...] * r_ref[...]


def kernel(left_input, right_input):
    B, F = left_input.shape
    rows = 1024
    return pl.pallas_call(
        _mul_body,
        grid=(B // rows,),
        in_specs=[
            pl.BlockSpec((rows, F), lambda i: (i, 0)),
            pl.BlockSpec((rows, F), lambda i: (i, 0)),
        ],
        out_specs=pl.BlockSpec((rows, F), lambda i: (i, 0)),
        out_shape=jax.ShapeDtypeStruct((B, F), left_input.dtype),
    )(left_input, right_input)
